# baseline (speedup 1.0000x reference)
import functools

import jax
import jax.numpy as jnp
from jax import lax
from jax.experimental import pallas as pl
from jax.experimental.pallas import tpu as pltpu
from jax.experimental.pallas import tpu_sc as plsc

D = 128
B = 16384
L = 50
N_ROWS = 777

NC = 2
NS = 16
NW = NC * NS
NPW = B // NW
SB = 128
NSB = NPW // SB
NPAIR = D // 2
PP = 16
NCP = NPAIR // PP


def _emb_pool_body(table_hbm, xt_hbm, ht_hbm, table_v, idxt_v, htsb_v):
    wid = lax.axis_index("s") * NC + lax.axis_index("c")
    base = wid * NPW
    pltpu.sync_copy(table_hbm, table_v)

    def sb_body(sb, carry):
        s0 = base + sb * SB
        pltpu.sync_copy(xt_hbm.at[:, pl.ds(s0, SB)], idxt_v)

        def b_body(b, c):
            b16 = b * 16
            for cp in range(NCP):
                pair0 = cp * PP

                init = tuple(jnp.zeros((16,), jnp.float32) for _ in range(2 * PP))
                colv = [jnp.full((16,), pair0 + p, jnp.int32) for p in range(PP)]

                @plsc.parallel_loop(0, L, unroll=4, carry=init)
                def accs(j, accs_in):
                    rowv = idxt_v[j, pl.ds(b16, 16)]
                    new = []
                    for p in range(PP):
                        g = plsc.load_gather(table_v, [colv[p], rowv])
                        lo, hi = plsc.unpack(
                            plsc.bitcast(g, jnp.bfloat16),
                            format=plsc.PackFormat.INTERLEAVED,
                            preferred_element_type=jnp.float32)
                        new.append(accs_in[2 * p] + lo)
                        new.append(accs_in[2 * p + 1] + hi)
                    return tuple(new)
                for k in range(2 * PP):
                    htsb_v[pair0 * 2 + k, pl.ds(b16, 16)] = accs[k]
            return c

        lax.fori_loop(0, SB // 16, b_body, 0)
        pltpu.sync_copy(htsb_v, ht_hbm.at[:, pl.ds(s0, SB)])
        return carry

    lax.fori_loop(0, NSB, sb_body, 0)


@functools.cache
def _emb_pool():
    mesh = plsc.VectorSubcoreMesh(core_axis_name="c", subcore_axis_name="s")
    return pl.kernel(
        _emb_pool_body,
        out_type=jax.ShapeDtypeStruct((D, B), jnp.float32),
        mesh=mesh,
        compiler_params=pltpu.CompilerParams(needs_layout_passes=False),
        scratch_types=[
            pltpu.VMEM((NPAIR, N_ROWS), jnp.int32),
            pltpu.VMEM((L, SB), jnp.int32),
            pltpu.VMEM((D, SB), jnp.float32),
        ],
    )


_SCALE = 256.0
_QUANT = 65536.0
_LOW = _QUANT / 2.0
_SHIFT = _QUANT * 5.0 + _LOW

MLP_BLK = 2048
_NBLK = B // MLP_BLK


def _mlp_body(ht_ref, n1_ref, n2_ref, ebt_ref, w1_ref, b1_ref, w2_ref, b2_ref,
              w3_ref, out_ref, pen_ref):
    i = pl.program_id(0)
    h = jnp.clip(ht_ref[...] + ebt_ref[...], 0.0, 1.0)
    t = (lax.dot_general(h, w1_ref[...], (((0,), (1,)), ((), ())),
                         preferred_element_type=jnp.float32) + b1_ref[...]) * _SCALE
    p1 = jnp.sum((jnp.maximum(jnp.abs(t) - _LOW * 0.5, 0.0) / _SCALE) ** 2)
    t = t + n1_ref[...]
    t = ((t + _SHIFT) % _QUANT - _QUANT / 2.0) / _SCALE
    t = jnp.clip(t, 0.0, 1.0)
    t = (lax.dot_general(t, w2_ref[...], (((1,), (1,)), ((), ())),
                         preferred_element_type=jnp.float32) + b2_ref[...]) * _SCALE
    p2 = jnp.sum((jnp.maximum(jnp.abs(t) - _LOW * 0.5, 0.0) / _SCALE) ** 2)
    t = t + n2_ref[...]
    t = ((t + _SHIFT) % _QUANT - _QUANT / 2.0) / _SCALE
    t = jnp.clip(t, 0.0, 1.0)
    out_ref[...] = jnp.sum(t * w3_ref[...], axis=1, keepdims=True)

    @pl.when(i == 0)
    def _():
        pen_ref[...] = jnp.zeros_like(pen_ref)

    pen_ref[...] += jnp.reshape(p1 + p2, (1, 1))

    @pl.when(i == _NBLK - 1)
    def _():
        pen_ref[...] = pen_ref[...] * (1.0 / (B * 32.0))


_mlp = pl.pallas_call(
    _mlp_body,
    grid=(_NBLK,),
    in_specs=[
        pl.BlockSpec((D, MLP_BLK), lambda i: (0, i)),
        pl.BlockSpec((MLP_BLK, 32), lambda i: (i, 0)),
        pl.BlockSpec((MLP_BLK, 32), lambda i: (i, 0)),
        pl.BlockSpec((D, 1), lambda i: (0, 0)),
        pl.BlockSpec((32, D), lambda i: (0, 0)),
        pl.BlockSpec((1, 32), lambda i: (0, 0)),
        pl.BlockSpec((32, 32), lambda i: (0, 0)),
        pl.BlockSpec((1, 32), lambda i: (0, 0)),
        pl.BlockSpec((1, 32), lambda i: (0, 0)),
    ],
    out_specs=[
        pl.BlockSpec((MLP_BLK, 1), lambda i: (i, 0)),
        pl.BlockSpec((1, 1), lambda i: (0, 0)),
    ],
    out_shape=[
        jax.ShapeDtypeStruct((B, 1), jnp.float32),
        jax.ShapeDtypeStruct((1, 1), jnp.float32),
    ],
)


def kernel(x, misc, tiles, coord, piece, row, col, tilecolor, zeros_param,
           emb_bias, W1, b1, W2, b2, W3, white_tile_mask, noise1, noise2):
    T = (tiles + coord + piece + row + col
         + tilecolor * white_tile_mask).reshape(12 * 8 * 8, D)
    table = jnp.concatenate([T, misc, zeros_param], axis=0)
    tb = lax.bitcast_convert_type(table.T.astype(jnp.bfloat16), jnp.uint16)
    tb = tb.astype(jnp.uint32)
    packed = lax.bitcast_convert_type(
        tb[0::2, :] | (tb[1::2, :] << 16), jnp.int32)
    ht = _emb_pool()(packed, x.T)
    ebt = emb_bias.reshape(D, 1)
    out, pen = _mlp(ht, noise1, noise2, ebt, W1, b1.reshape(1, 32), W2,
                    b2.reshape(1, 32), W3)
    return out, pen[0, 0]


# speedup vs baseline: 1.7023x; 1.7023x over previous
import functools

import jax
import jax.numpy as jnp
from jax import lax
from jax.experimental import pallas as pl
from jax.experimental.pallas import tpu as pltpu
from jax.experimental.pallas import tpu_sc as plsc

D = 128
B = 16384
L = 50
N_ROWS = 777

NC = 2
NS = 16
NW = NC * NS
NPW = B // NW
SB = 128
NSB = NPW // SB
NPAIR = D // 2
PP = 16
NCP = NPAIR // PP


def _emb_pool_body(table_hbm, xt_hbm, ht_hbm, table_v, idxt_v, htsb_v):
    wid = lax.axis_index("s") * NC + lax.axis_index("c")
    base = wid * NPW
    pltpu.sync_copy(table_hbm, table_v)

    def sb_body(sb, carry):
        s0 = base + sb * SB
        pltpu.sync_copy(xt_hbm.at[:, pl.ds(s0, SB)], idxt_v)

        def b_body(b, c):
            b16 = b * 16
            for cp in range(NCP):
                pair0 = cp * PP

                init = tuple(jnp.zeros((16,), jnp.float32) for _ in range(2 * PP))
                colv = [jnp.full((16,), pair0 + p, jnp.int32) for p in range(PP)]

                @plsc.parallel_loop(0, L, unroll=2, carry=init)
                def accs(j, accs_in):
                    rowv = idxt_v[j, pl.ds(b16, 16)]
                    new = []
                    for p in range(PP):
                        g = plsc.load_gather(table_v, [colv[p], rowv])
                        lo, hi = plsc.unpack(
                            plsc.bitcast(g, jnp.bfloat16),
                            format=plsc.PackFormat.INTERLEAVED,
                            preferred_element_type=jnp.float32)
                        new.append(accs_in[2 * p] + lo)
                        new.append(accs_in[2 * p + 1] + hi)
                    return tuple(new)
                for k in range(2 * PP):
                    htsb_v[pair0 * 2 + k, pl.ds(b16, 16)] = accs[k]
            return c

        lax.fori_loop(0, SB // 16, b_body, 0)
        pltpu.sync_copy(htsb_v, ht_hbm.at[:, pl.ds(s0, SB)])
        return carry

    lax.fori_loop(0, NSB, sb_body, 0)


@functools.cache
def _emb_pool():
    mesh = plsc.VectorSubcoreMesh(core_axis_name="c", subcore_axis_name="s")
    return pl.kernel(
        _emb_pool_body,
        out_type=jax.ShapeDtypeStruct((D, B), jnp.float32),
        mesh=mesh,
        compiler_params=pltpu.CompilerParams(needs_layout_passes=False),
        scratch_types=[
            pltpu.VMEM((NPAIR, N_ROWS), jnp.int32),
            pltpu.VMEM((L, SB), jnp.int32),
            pltpu.VMEM((D, SB), jnp.float32),
        ],
    )


_SCALE = 256.0
_QUANT = 65536.0
_LOW = _QUANT / 2.0
_SHIFT = _QUANT * 5.0 + _LOW

MLP_BLK = 2048
_NBLK = B // MLP_BLK


def _mlp_body(ht_ref, n1_ref, n2_ref, ebt_ref, w1_ref, b1_ref, w2_ref, b2_ref,
              w3_ref, out_ref, pen_ref):
    i = pl.program_id(0)
    h = jnp.clip(ht_ref[...] + ebt_ref[...], 0.0, 1.0)
    t = (lax.dot_general(h, w1_ref[...], (((0,), (1,)), ((), ())),
                         preferred_element_type=jnp.float32) + b1_ref[...]) * _SCALE
    p1 = jnp.sum((jnp.maximum(jnp.abs(t) - _LOW * 0.5, 0.0) / _SCALE) ** 2)
    t = t + n1_ref[...]
    t = ((t + _SHIFT) % _QUANT - _QUANT / 2.0) / _SCALE
    t = jnp.clip(t, 0.0, 1.0)
    t = (lax.dot_general(t, w2_ref[...], (((1,), (1,)), ((), ())),
                         preferred_element_type=jnp.float32) + b2_ref[...]) * _SCALE
    p2 = jnp.sum((jnp.maximum(jnp.abs(t) - _LOW * 0.5, 0.0) / _SCALE) ** 2)
    t = t + n2_ref[...]
    t = ((t + _SHIFT) % _QUANT - _QUANT / 2.0) / _SCALE
    t = jnp.clip(t, 0.0, 1.0)
    out_ref[...] = jnp.sum(t * w3_ref[...], axis=1, keepdims=True)

    @pl.when(i == 0)
    def _():
        pen_ref[...] = jnp.zeros_like(pen_ref)

    pen_ref[...] += jnp.reshape(p1 + p2, (1, 1))

    @pl.when(i == _NBLK - 1)
    def _():
        pen_ref[...] = pen_ref[...] * (1.0 / (B * 32.0))


_mlp = pl.pallas_call(
    _mlp_body,
    grid=(_NBLK,),
    in_specs=[
        pl.BlockSpec((D, MLP_BLK), lambda i: (0, i)),
        pl.BlockSpec((MLP_BLK, 32), lambda i: (i, 0)),
        pl.BlockSpec((MLP_BLK, 32), lambda i: (i, 0)),
        pl.BlockSpec((D, 1), lambda i: (0, 0)),
        pl.BlockSpec((32, D), lambda i: (0, 0)),
        pl.BlockSpec((1, 32), lambda i: (0, 0)),
        pl.BlockSpec((32, 32), lambda i: (0, 0)),
        pl.BlockSpec((1, 32), lambda i: (0, 0)),
        pl.BlockSpec((1, 32), lambda i: (0, 0)),
    ],
    out_specs=[
        pl.BlockSpec((MLP_BLK, 1), lambda i: (i, 0)),
        pl.BlockSpec((1, 1), lambda i: (0, 0)),
    ],
    out_shape=[
        jax.ShapeDtypeStruct((B, 1), jnp.float32),
        jax.ShapeDtypeStruct((1, 1), jnp.float32),
    ],
)


def kernel(x, misc, tiles, coord, piece, row, col, tilecolor, zeros_param,
           emb_bias, W1, b1, W2, b2, W3, white_tile_mask, noise1, noise2):
    T = (tiles + coord + piece + row + col
         + tilecolor * white_tile_mask).reshape(12 * 8 * 8, D)
    table = jnp.concatenate([T, misc, zeros_param], axis=0)
    tb = lax.bitcast_convert_type(table.T.astype(jnp.bfloat16), jnp.uint16)
    tb = tb.astype(jnp.uint32)
    packed = lax.bitcast_convert_type(
        tb[0::2, :] | (tb[1::2, :] << 16), jnp.int32)
    ht = _emb_pool()(packed, x.T)
    ebt = emb_bias.reshape(D, 1)
    out, pen = _mlp(ht, noise1, noise2, ebt, W1, b1.reshape(1, 32), W2,
                    b2.reshape(1, 32), W3)
    return out, pen[0, 0]


# speedup vs baseline: 1.8610x; 1.0933x over previous
import functools

import jax
import jax.numpy as jnp
from jax import lax
from jax.experimental import pallas as pl
from jax.experimental.pallas import tpu as pltpu
from jax.experimental.pallas import tpu_sc as plsc

D = 128
B = 16384
L = 50
N_ROWS = 777

NC = 2
NS = 16
NW = NC * NS
NPW = B // NW
SB = 128
NSB = NPW // SB
NPAIR = D // 2
PP = 8
NCP = NPAIR // PP


def _emb_pool_body(table_hbm, xt_hbm, ht_hbm, table_v, idxt_v, htsb_v):
    wid = lax.axis_index("s") * NC + lax.axis_index("c")
    base = wid * NPW
    pltpu.sync_copy(table_hbm, table_v)

    def sb_body(sb, carry):
        s0 = base + sb * SB
        pltpu.sync_copy(xt_hbm.at[:, pl.ds(s0, SB)], idxt_v)

        iota16 = jnp.arange(16, dtype=jnp.int32)

        def b_body(b, c):
            b16 = b * 16
            samplev = iota16 + b16
            for cp in range(NCP):
                k0 = cp * PP
                diag = [(iota16 + (k0 + kk)) % NPAIR for kk in range(PP)]
                init = tuple(jnp.zeros((16,), jnp.float32) for _ in range(2 * PP))

                @plsc.parallel_loop(0, L, unroll=2, carry=init)
                def accs(j, accs_in):
                    rowv = idxt_v[j, pl.ds(b16, 16)]
                    new = []
                    for kk in range(PP):
                        g = plsc.load_gather(table_v, [rowv, diag[kk]])
                        lo, hi = plsc.unpack(
                            plsc.bitcast(g, jnp.bfloat16),
                            format=plsc.PackFormat.INTERLEAVED,
                            preferred_element_type=jnp.float32)
                        new.append(accs_in[2 * kk] + lo)
                        new.append(accs_in[2 * kk + 1] + hi)
                    return tuple(new)

                for kk in range(PP):
                    colv = diag[kk] * 2
                    plsc.store_scatter(htsb_v, [colv, samplev], accs[2 * kk])
                    plsc.store_scatter(htsb_v, [colv + 1, samplev], accs[2 * kk + 1])
            return c

        lax.fori_loop(0, SB // 16, b_body, 0)
        pltpu.sync_copy(htsb_v, ht_hbm.at[:, pl.ds(s0, SB)])
        return carry

    lax.fori_loop(0, NSB, sb_body, 0)


@functools.cache
def _emb_pool():
    mesh = plsc.VectorSubcoreMesh(core_axis_name="c", subcore_axis_name="s")
    return pl.kernel(
        _emb_pool_body,
        out_type=jax.ShapeDtypeStruct((D, B), jnp.float32),
        mesh=mesh,
        compiler_params=pltpu.CompilerParams(needs_layout_passes=False),
        scratch_types=[
            pltpu.VMEM((N_ROWS, NPAIR), jnp.int32),
            pltpu.VMEM((L, SB), jnp.int32),
            pltpu.VMEM((D, SB), jnp.float32),
        ],
    )


_SCALE = 256.0
_QUANT = 65536.0
_LOW = _QUANT / 2.0
_SHIFT = _QUANT * 5.0 + _LOW

MLP_BLK = 2048
_NBLK = B // MLP_BLK


def _mlp_body(ht_ref, n1_ref, n2_ref, ebt_ref, w1_ref, b1_ref, w2_ref, b2_ref,
              w3_ref, out_ref, pen_ref):
    i = pl.program_id(0)
    h = jnp.clip(ht_ref[...] + ebt_ref[...], 0.0, 1.0)
    t = (lax.dot_general(h, w1_ref[...], (((0,), (1,)), ((), ())),
                         preferred_element_type=jnp.float32) + b1_ref[...]) * _SCALE
    p1 = jnp.sum((jnp.maximum(jnp.abs(t) - _LOW * 0.5, 0.0) / _SCALE) ** 2)
    t = t + n1_ref[...]
    t = ((t + _SHIFT) % _QUANT - _QUANT / 2.0) / _SCALE
    t = jnp.clip(t, 0.0, 1.0)
    t = (lax.dot_general(t, w2_ref[...], (((1,), (1,)), ((), ())),
                         preferred_element_type=jnp.float32) + b2_ref[...]) * _SCALE
    p2 = jnp.sum((jnp.maximum(jnp.abs(t) - _LOW * 0.5, 0.0) / _SCALE) ** 2)
    t = t + n2_ref[...]
    t = ((t + _SHIFT) % _QUANT - _QUANT / 2.0) / _SCALE
    t = jnp.clip(t, 0.0, 1.0)
    out_ref[...] = jnp.sum(t * w3_ref[...], axis=1, keepdims=True)

    @pl.when(i == 0)
    def _():
        pen_ref[...] = jnp.zeros_like(pen_ref)

    pen_ref[...] += jnp.reshape(p1 + p2, (1, 1))

    @pl.when(i == _NBLK - 1)
    def _():
        pen_ref[...] = pen_ref[...] * (1.0 / (B * 32.0))


_mlp = pl.pallas_call(
    _mlp_body,
    grid=(_NBLK,),
    in_specs=[
        pl.BlockSpec((D, MLP_BLK), lambda i: (0, i)),
        pl.BlockSpec((MLP_BLK, 32), lambda i: (i, 0)),
        pl.BlockSpec((MLP_BLK, 32), lambda i: (i, 0)),
        pl.BlockSpec((D, 1), lambda i: (0, 0)),
        pl.BlockSpec((32, D), lambda i: (0, 0)),
        pl.BlockSpec((1, 32), lambda i: (0, 0)),
        pl.BlockSpec((32, 32), lambda i: (0, 0)),
        pl.BlockSpec((1, 32), lambda i: (0, 0)),
        pl.BlockSpec((1, 32), lambda i: (0, 0)),
    ],
    out_specs=[
        pl.BlockSpec((MLP_BLK, 1), lambda i: (i, 0)),
        pl.BlockSpec((1, 1), lambda i: (0, 0)),
    ],
    out_shape=[
        jax.ShapeDtypeStruct((B, 1), jnp.float32),
        jax.ShapeDtypeStruct((1, 1), jnp.float32),
    ],
)


def kernel(x, misc, tiles, coord, piece, row, col, tilecolor, zeros_param,
           emb_bias, W1, b1, W2, b2, W3, white_tile_mask, noise1, noise2):
    T = (tiles + coord + piece + row + col
         + tilecolor * white_tile_mask).reshape(12 * 8 * 8, D)
    table = jnp.concatenate([T, misc, zeros_param], axis=0)
    tb = lax.bitcast_convert_type(table.astype(jnp.bfloat16), jnp.uint16)
    tb = tb.astype(jnp.uint32)
    packed = lax.bitcast_convert_type(
        tb[:, 0::2] | (tb[:, 1::2] << 16), jnp.int32)
    ht = _emb_pool()(packed, x.T)
    ebt = emb_bias.reshape(D, 1)
    out, pen = _mlp(ht, noise1, noise2, ebt, W1, b1.reshape(1, 32), W2,
                    b2.reshape(1, 32), W3)
    return out, pen[0, 0]
